# baseline (speedup 1.0000x reference)
import functools

import jax
import jax.numpy as jnp
from jax import lax
from jax.experimental import pallas as pl
from jax.experimental.pallas import tpu as pltpu
from jax.experimental.pallas import tpu_sc as plsc

_NUM_SNPS = 100000
_NUM_IMPACTS = 16
_EMB = 16
_BATCH = 1024

_ROWS = 2000
_GRID = _NUM_SNPS // _ROWS

_SC_PAD = 102400
_PER_W = _SC_PAD // 32


def _head_body(emb_ref, wpt_ref, bp_ref, gamma_ref, beta_ref, wsb_ref,
               bsbb_ref, tab_ref):
    h = jnp.dot(emb_ref[...], wpt_ref[...],
                preferred_element_type=jnp.float32) + bp_ref[...]
    mu = jnp.mean(h, axis=-1, keepdims=True)
    var = jnp.mean((h - mu) ** 2, axis=-1, keepdims=True)
    h = (h - mu) / jnp.sqrt(var + 1e-5) * gamma_ref[...] + beta_ref[...]
    h = jnp.maximum(h, 0.0)
    tab = jnp.dot(h, wsb_ref[...],
                  preferred_element_type=jnp.float32) + bsbb_ref[...]
    tab_ref[...] = tab * 0.5


def _sc_gather_body(stab_hbm, btab_hbm, idx_hbm, sout_hbm, bout_hbm,
                    stab_v, btab_v, idx_v, s_v, b_v):
    wid = lax.axis_index("s") * 2 + lax.axis_index("c")
    base = wid * _PER_W
    pltpu.sync_copy(stab_hbm, stab_v)
    pltpu.sync_copy(btab_hbm, btab_v)
    pltpu.sync_copy(idx_hbm.at[pl.ds(base, _PER_W)], idx_v)

    def body(i, carry):
        iv = idx_v[pl.ds(i * 16, 16)]
        s_v[pl.ds(i * 16, 16)] = plsc.load_gather(stab_v, [iv])
        b_v[pl.ds(i * 16, 16)] = plsc.load_gather(btab_v, [iv])
        return carry

    lax.fori_loop(0, _PER_W // 16, body, 0)
    pltpu.sync_copy(s_v, sout_hbm.at[pl.ds(base, _PER_W)])
    pltpu.sync_copy(b_v, bout_hbm.at[pl.ds(base, _PER_W)])


def _dense_body(s_ref, b_ref, x_ref, o_ref):
    xx = x_ref[...]
    ss = s_ref[...]
    bb = b_ref[...]
    o_ref[...] = xx + xx * jnp.tanh(xx * ss + bb)


def kernel(x, impact_indices, emb, Wp, bp, gamma, beta, ws, bs, wb, bb):
    wpt = Wp.T
    wsb = jnp.concatenate([ws, wb], axis=1)
    bsbb = jnp.concatenate([bs, bb]).reshape(1, 2)

    tab = pl.pallas_call(
        _head_body,
        out_shape=jax.ShapeDtypeStruct((_NUM_IMPACTS, 2), jnp.float32),
    )(emb, wpt, bp.reshape(1, _EMB), gamma.reshape(1, _EMB),
      beta.reshape(1, _EMB), wsb, bsbb)

    idx_pad = jnp.pad(impact_indices, (0, _SC_PAD - _NUM_SNPS))

    sc_gather = functools.partial(
        pl.kernel,
        out_type=(jax.ShapeDtypeStruct((_SC_PAD,), jnp.float32),
                  jax.ShapeDtypeStruct((_SC_PAD,), jnp.float32)),
        mesh=plsc.VectorSubcoreMesh(core_axis_name="c", subcore_axis_name="s"),
        scratch_types=[
            pltpu.VMEM((_NUM_IMPACTS,), jnp.float32),
            pltpu.VMEM((_NUM_IMPACTS,), jnp.float32),
            pltpu.VMEM((_PER_W,), jnp.int32),
            pltpu.VMEM((_PER_W,), jnp.float32),
            pltpu.VMEM((_PER_W,), jnp.float32),
        ],
        compiler_params=pltpu.CompilerParams(needs_layout_passes=False),
    )(_sc_gather_body)
    s_row, b_row = sc_gather(tab[:, 0], tab[:, 1], idx_pad)

    s_col = s_row[:_NUM_SNPS].reshape(_NUM_SNPS, 1)
    b_col = b_row[:_NUM_SNPS].reshape(_NUM_SNPS, 1)
    xt = x.T

    out_t = pl.pallas_call(
        _dense_body,
        grid=(_GRID,),
        in_specs=[
            pl.BlockSpec((_ROWS, 1), lambda j: (j, 0)),
            pl.BlockSpec((_ROWS, 1), lambda j: (j, 0)),
            pl.BlockSpec((_ROWS, _BATCH), lambda j: (j, 0)),
        ],
        out_specs=pl.BlockSpec((_ROWS, _BATCH), lambda j: (j, 0)),
        out_shape=jax.ShapeDtypeStruct((_NUM_SNPS, _BATCH), jnp.float32),
        compiler_params=pltpu.CompilerParams(
            dimension_semantics=("parallel",)),
    )(s_col, b_col, xt)
    return out_t.T


# speedup vs baseline: 1.1776x; 1.1776x over previous
import jax
import jax.numpy as jnp
from jax.experimental import pallas as pl
from jax.experimental.pallas import tpu as pltpu

_NUM_SNPS = 100000
_NUM_IMPACTS = 16
_EMB = 16
_BATCH = 1024

_ROWS = 2000
_GRID = _NUM_SNPS // _ROWS


def _head_body(emb_ref, wpt_ref, bp_ref, gamma_ref, beta_ref, wsb_ref,
               bsbb_ref, idx_ref, sb_ref):
    h = jnp.dot(emb_ref[...], wpt_ref[...],
                preferred_element_type=jnp.float32) + bp_ref[...]
    mu = jnp.mean(h, axis=-1, keepdims=True)
    var = jnp.mean((h - mu) ** 2, axis=-1, keepdims=True)
    h = (h - mu) / jnp.sqrt(var + 1e-5) * gamma_ref[...] + beta_ref[...]
    h = jnp.maximum(h, 0.0)
    tab = jnp.dot(h, wsb_ref[...],
                  preferred_element_type=jnp.float32) + bsbb_ref[...]
    idx = idx_ref[...]
    ss = jnp.full(idx.shape, tab[0, 0] * 0.5, jnp.float32)
    bb = jnp.full(idx.shape, tab[0, 1] * 0.5, jnp.float32)
    for k in range(1, _NUM_IMPACTS):
        m = idx == k
        ss = jnp.where(m, tab[k, 0] * 0.5, ss)
        bb = jnp.where(m, tab[k, 1] * 0.5, bb)
    sb_ref[0:1, :] = ss
    sb_ref[1:2, :] = bb


def _dense_body(s_ref, b_ref, x_ref, o_ref):
    xx = x_ref[...]
    ss = s_ref[...]
    bb = b_ref[...]
    o_ref[...] = xx + xx * jnp.tanh(xx * ss + bb)


def kernel(x, impact_indices, emb, Wp, bp, gamma, beta, ws, bs, wb, bb):
    wpt = Wp.T
    wsb = jnp.concatenate([ws, wb], axis=1)
    bsbb = jnp.concatenate([bs, bb]).reshape(1, 2)
    idx = impact_indices.reshape(1, _NUM_SNPS)

    sb = pl.pallas_call(
        _head_body,
        out_shape=jax.ShapeDtypeStruct((2, _NUM_SNPS), jnp.float32),
    )(emb, wpt, bp.reshape(1, _EMB), gamma.reshape(1, _EMB),
      beta.reshape(1, _EMB), wsb, bsbb, idx)

    s_col = sb[0].reshape(_NUM_SNPS, 1)
    b_col = sb[1].reshape(_NUM_SNPS, 1)
    xt = x.T

    out_t = pl.pallas_call(
        _dense_body,
        grid=(_GRID,),
        in_specs=[
            pl.BlockSpec((_ROWS, 1), lambda j: (j, 0)),
            pl.BlockSpec((_ROWS, 1), lambda j: (j, 0)),
            pl.BlockSpec((_ROWS, _BATCH), lambda j: (j, 0)),
        ],
        out_specs=pl.BlockSpec((_ROWS, _BATCH), lambda j: (j, 0)),
        out_shape=jax.ShapeDtypeStruct((_NUM_SNPS, _BATCH), jnp.float32),
        compiler_params=pltpu.CompilerParams(
            dimension_semantics=("parallel",)),
    )(s_col, b_col, xt)
    return out_t.T
